# Initial kernel scaffold; baseline (speedup 1.0000x reference)
import jax
import jax.numpy as jnp
from jax.experimental import pallas as pl


def kernel(x, edge_index, W0, b0, W1, b1):
    raise NotImplementedError("write your pallas kernel here")


# speedup vs baseline: 13.1355x; 13.1355x over previous
import functools

import jax
import jax.numpy as jnp
from jax import lax
from jax.experimental import pallas as pl
from jax.experimental.pallas import tpu as pltpu
from jax.experimental.pallas import tpu_sc as plsc

NC = 2
NS = 16
NW = NC * NS
CH = 128


def _mesh():
    return plsc.VectorSubcoreMesh(
        core_axis_name="c", subcore_axis_name="s", num_cores=NC, num_subcores=NS
    )


def _sc_degree(col_r, np_, kch):
    rps = np_ // NS
    zr = 40
    nz = rps // zr

    def body(col_hbm, out_hbm, idx_v, ones_v, zeros_v, cnt_sh):
        cid = lax.axis_index("c")
        sid = lax.axis_index("s")
        wid = sid * NC + cid
        for i in range(CH):
            ones_v[i, :] = jnp.ones((16,), jnp.float32)
        for i in range(zr):
            zeros_v[i, :] = jnp.zeros((16,), jnp.float32)
        base = sid * rps
        for k in range(nz):
            pltpu.sync_copy(zeros_v, cnt_sh.at[pl.ds(base + k * zr, zr)])
        plsc.subcore_barrier()
        pltpu.sync_copy(col_hbm.at[wid], idx_v)

        def step(j, carry):
            pltpu.sync_copy(ones_v, cnt_sh.at[idx_v.at[j]], add=True)
            return carry

        lax.fori_loop(0, kch, step, 0)
        plsc.subcore_barrier()
        pltpu.sync_copy(cnt_sh.at[pl.ds(base, rps)],
                        out_hbm.at[cid, pl.ds(base, rps)])

    return pl.kernel(
        body,
        out_type=jax.ShapeDtypeStruct((NC, np_, 16), jnp.float32),
        mesh=_mesh(),
        scratch_types=[
            pltpu.VMEM((kch, CH), jnp.int32),
            pltpu.VMEM((CH, 16), jnp.float32),
            pltpu.VMEM((zr, 16), jnp.float32),
            pltpu.VMEM_SHARED((np_, 16), jnp.float32),
        ],
    )(col_r)


def _sc_aggregate(hp, row_r, col_r, np_, kch):
    d = hp.shape[1]
    rps = np_ // NS

    def body(hp_hbm, row_hbm, col_hbm, out_hbm, rowi_v, coli_v, gbuf, agg_sh, sem):
        cid = lax.axis_index("c")
        sid = lax.axis_index("s")
        wid = sid * NC + cid
        base = sid * rps
        pltpu.sync_copy(hp_hbm.at[pl.ds(base, rps)], agg_sh.at[pl.ds(base, rps)])
        plsc.subcore_barrier()
        pltpu.sync_copy(row_hbm.at[wid], rowi_v)
        pltpu.sync_copy(col_hbm.at[wid], coli_v)

        def step(j, carry):
            pltpu.async_copy(hp_hbm.at[rowi_v.at[j]], gbuf, sem).wait()
            pltpu.sync_copy(gbuf, agg_sh.at[coli_v.at[j]], add=True)
            return carry

        lax.fori_loop(0, kch, step, 0)
        plsc.subcore_barrier()
        pltpu.sync_copy(agg_sh.at[pl.ds(base, rps)],
                        out_hbm.at[cid, pl.ds(base, rps)])

    return pl.kernel(
        body,
        out_type=jax.ShapeDtypeStruct((NC, np_, d), jnp.float32),
        mesh=_mesh(),
        scratch_types=[
            pltpu.VMEM((kch, CH), jnp.int32),
            pltpu.VMEM((kch, CH), jnp.int32),
            pltpu.VMEM((CH, d), jnp.float32),
            pltpu.VMEM_SHARED((np_, d), jnp.float32),
            pltpu.SemaphoreType.DMA,
        ],
    )(hp, row_r, col_r)


def _tc_h0(x_p, w0, cnt):
    np_, d = x_p.shape
    h = w0.shape[1]

    def body(x_ref, w_ref, cnt_ref, hp_ref, dinv_ref):
        c = cnt_ref[0, :, 0:1] + cnt_ref[1, :, 0:1]
        dinv = lax.rsqrt(c + 1.0)
        y = jnp.dot(x_ref[...], w_ref[...], preferred_element_type=jnp.float32)
        hp_ref[...] = y * dinv
        dinv_ref[...] = dinv

    return pl.pallas_call(
        body,
        out_shape=[
            jax.ShapeDtypeStruct((np_, h), jnp.float32),
            jax.ShapeDtypeStruct((np_, 1), jnp.float32),
        ],
    )(x_p, w0, cnt)


def _tc_mid(agg, hp0, dinv, b0, w1):
    np_, d = hp0.shape
    c = w1.shape[1]

    def body(agg_ref, hp_ref, dinv_ref, b_ref, w_ref, out_ref):
        s = agg_ref[0] + agg_ref[1] - hp_ref[...]
        z = jnp.maximum(s * dinv_ref[...] + b_ref[...], 0.0)
        out_ref[...] = (
            jnp.dot(z, w_ref[...], preferred_element_type=jnp.float32)
            * dinv_ref[...]
        )

    return pl.pallas_call(
        body,
        out_shape=jax.ShapeDtypeStruct((np_, c), jnp.float32),
    )(agg, hp0, dinv, b0, w1)


def _tc_out(agg, hp1, dinv, b1, n):
    d = hp1.shape[1]

    def body(agg_ref, hp_ref, dinv_ref, b_ref, out_ref):
        s = agg_ref[0, :n, :] + agg_ref[1, :n, :] - hp_ref[:n, :]
        out_ref[...] = s * dinv_ref[:n, :] + b_ref[...]

    return pl.pallas_call(
        body,
        out_shape=jax.ShapeDtypeStruct((n, d), jnp.float32),
    )(agg, hp1, dinv, b1)


def kernel(x, edge_index, W0, b0, W1, b1):
    n, d = x.shape
    e = edge_index.shape[1]
    np_ = ((n + 1 + NS * 8 - 1) // (NS * 8)) * (NS * 8)
    kch = -(-e // (NW * CH))
    ep = NW * kch * CH

    row = edge_index[0]
    col = edge_index[1]
    pad = ep - e
    row_r = jnp.concatenate([row, jnp.zeros((pad,), row.dtype)]).reshape(NW, kch, CH)
    col_r = jnp.concatenate([col, jnp.full((pad,), n, col.dtype)]).reshape(NW, kch, CH)
    x_p = jnp.pad(x, ((0, np_ - n), (0, 0)))

    cnt = _sc_degree(col_r, np_, kch)
    hp0, dinv = _tc_h0(x_p, W0, cnt)
    agg0 = _sc_aggregate(hp0, row_r, col_r, np_, kch)
    hp1 = _tc_mid(agg0, hp0, dinv, b0.reshape(1, -1), W1)
    agg1 = _sc_aggregate(hp1, row_r, col_r, np_, kch)
    return _tc_out(agg1, hp1, dinv, b1.reshape(1, -1), n)
